# baseline (speedup 1.0000x reference)
import functools
import numpy as np
import jax
import jax.numpy as jnp
from jax import lax
from jax.experimental import pallas as pl
from jax.experimental.pallas import tpu as pltpu

_NUM_CLASSES = 90
_K = 5000
_W = 5120
_CH1 = 40960
_CH2 = 4096
_NBITS = 4
_NPASS = 8
_NBIN = 16
_IMIN = np.int32(-2**31)
_NEG = np.float32(-np.inf)


def _key_of(x):
    i = lax.bitcast_convert_type(x, jnp.int32)
    return jnp.where(i < 0, jnp.bitwise_not(i), jnp.bitwise_or(i, _IMIN))


def _k1_body(x_ref, t_ref, r_ref, hist, pref_s, rank_s):
    p = pl.program_id(0)
    b = pl.program_id(1)
    c = pl.program_id(2)
    nch = pl.num_programs(2)

    @pl.when(jnp.logical_and(p == 0, c == 0))
    def _():
        pref_s[b] = jnp.int32(0)
        rank_s[b] = jnp.int32(_K)

    @pl.when(c == 0)
    def _():
        for k in range(_NBIN):
            hist[k] = jnp.int32(0)

    shift = jnp.int32(28) - jnp.int32(_NBITS) * p
    x = x_ref[0, 0, 0, :].reshape(_CH1 // 128, 128)
    key = _key_of(x)
    pref = pref_s[b]
    shv = jnp.broadcast_to(shift, key.shape)
    match = lax.shift_right_logical(
        lax.shift_right_logical(jnp.bitwise_xor(key, jnp.broadcast_to(pref, key.shape)), shv),
        jnp.full_like(key, _NBITS)) == 0
    binv = jnp.bitwise_and(lax.shift_right_logical(key, shv), _NBIN - 1)
    for j in range(_NBIN):
        cnt_j = jnp.sum(jnp.where(jnp.logical_and(match, binv == j), 1, 0).astype(jnp.int32))
        hist[j] = hist[j] + cnt_j

    @pl.when(c == nch - 1)
    def _():
        r = rank_s[b]
        acc = jnp.int32(0)
        jstar = jnp.int32(-1)
        above = jnp.int32(0)
        for k in range(_NBIN - 1, -1, -1):
            hk = hist[k]
            acc_new = acc + hk
            take = jnp.logical_and(acc_new >= r, jstar < 0)
            jstar = jnp.where(take, jnp.int32(k), jstar)
            above = jnp.where(take, acc, above)
            acc = acc_new
        pref_new = jnp.bitwise_or(pref, lax.shift_left(jstar, shift))
        r_new = r - above
        pref_s[b] = pref_new
        rank_s[b] = r_new
        t_ref[...] = jnp.full(t_ref.shape, pref_new, jnp.int32)
        r_ref[...] = jnp.full(r_ref.shape, r_new, jnp.int32)


def _k2_body(x_ref, t_ref, r_ref, oi_ref, ov_ref, st):
    c = pl.program_id(1)

    @pl.when(c == 0)
    def _():
        st[0] = jnp.int32(0)
        st[1] = jnp.int32(0)

        def initb(j, carry):
            oi_ref[0, 0, _K + j] = jnp.int32(-1)
            ov_ref[0, 0, _K + j] = _NEG
            return carry

        lax.fori_loop(0, _W - _K, initb, jnp.int32(0))

    x = x_ref[0, 0, 0, :].reshape(_CH2 // 128, 128)
    key = _key_of(x)
    ks = jnp.bitwise_xor(key, _IMIN)
    tb = jnp.broadcast_to(t_ref[0][0:1, :], key.shape)
    ts = jnp.bitwise_xor(tb, _IMIN)
    gt = ks > ts
    eq = key == tb
    cnt_gt = jnp.sum(gt.astype(jnp.int32))
    cnt_eq = jnp.sum(eq.astype(jnp.int32))
    r0 = jnp.max(r_ref[0])
    ties = st[1]
    n_eq = jnp.clip(r0 - ties, 0, cnt_eq)
    cnt = cnt_gt + n_eq
    base = st[0]

    rows = _CH2 // 128
    lin = (lax.broadcasted_iota(jnp.int32, (rows, 128), 0) * 128
           + lax.broadcasted_iota(jnp.int32, (rows, 128), 1))
    xm0 = jnp.where(jnp.logical_or(gt, eq), x, _NEG)

    def body(j, xm):
        m = jnp.max(xm)
        iloc = jnp.min(jnp.where(xm == m, lin, jnp.int32(10**9)))
        pos = base + j
        oi_ref[0, 0, pos] = c * _CH2 + iloc
        ov_ref[0, 0, pos] = m
        return jnp.where(lin == iloc, _NEG, xm)

    lax.fori_loop(0, cnt, body, xm0)
    st[0] = base + cnt
    st[1] = ties + n_eq


def _k3_body(a_pad, tab_ref, v_ref, i_ref, o_ref, scr):
    rows = _W // 128
    logit = v_ref[0, 0, :].reshape(rows, 128)
    idx = i_ref[0, 0, :].reshape(rows, 128)
    idxc = jnp.maximum(idx, 0)
    a = idxc // _NUM_CLASSES
    cls_f = (idxc - a * _NUM_CLASSES).astype(jnp.float32)

    ai = a.reshape(1, _W)
    kc = min(512, a_pad)
    r_acc = jnp.zeros((8, _W), jnp.float32)
    for k0 in range(0, a_pad, kc):
        rid = lax.broadcasted_iota(jnp.int32, (kc, _W), 0) + k0
        oh = (rid == jnp.broadcast_to(ai, (kc, _W))).astype(jnp.float32)
        r_acc = r_acc + jax.lax.dot_general(
            tab_ref[0, :, k0:k0 + kc], oh, (((1,), (0,)), ((), ())),
            preferred_element_type=jnp.float32)

    ty = r_acc[0].reshape(rows, 128)
    tx = r_acc[1].reshape(rows, 128)
    th = r_acc[2].reshape(rows, 128)
    tw = r_acc[3].reshape(rows, 128)
    ya1 = r_acc[4].reshape(rows, 128)
    xa1 = r_acc[5].reshape(rows, 128)
    ya2 = r_acc[6].reshape(rows, 128)
    xa2 = r_acc[7].reshape(rows, 128)

    yca = (ya1 + ya2) * 0.5
    xca = (xa1 + xa2) * 0.5
    ha = ya2 - ya1
    wa = xa2 - xa1
    w = jnp.exp(tw) * wa
    h = jnp.exp(th) * ha
    yc = ty * ha + yca
    xc = tx * wa + xca
    x1 = xc - w * 0.5
    y1 = yc - h * 0.5
    x2 = xc + w * 0.5
    y2 = yc + h * 0.5
    offs = cls_f * 1e4
    scr[0] = x1 + offs
    scr[1] = y1 + offs
    scr[2] = x2 + offs
    scr[3] = y2 + offs
    scr[4] = (x2 - x1) * (y2 - y1)
    scr[5] = x1
    scr[6] = y1
    scr[7] = x2
    scr[8] = y2
    scr[9] = cls_f
    scr[10] = 1.0 / (1.0 + jnp.exp(-logit))
    scr[11] = logit

    lin = (lax.broadcasted_iota(jnp.int32, (rows, 128), 0) * 128
           + lax.broadcasted_iota(jnp.int32, (rows, 128), 1))
    lane = lax.broadcasted_iota(jnp.int32, (1, 128), 1)

    def body(j, carry):
        sc = scr[11]
        m = jnp.max(sc)
        il = jnp.min(jnp.where(sc == m, lin, jnp.int32(10**9)))
        pickm = lin == il

        def rd(k):
            return jnp.sum(jnp.where(pickm, scr[k], 0.0))

        bb0, bb1, bb2, bb3 = rd(0), rd(1), rd(2), rd(3)
        area_i = rd(4)
        ox1, oy1, ox2, oy2 = rd(5), rd(6), rd(7), rd(8)
        cls_i = rd(9)
        sig_i = rd(10)

        xx1 = jnp.maximum(bb0, scr[0])
        yy1 = jnp.maximum(bb1, scr[1])
        xx2 = jnp.minimum(bb2, scr[2])
        yy2 = jnp.minimum(bb3, scr[3])
        inter = jnp.clip(xx2 - xx1, 0.0, None) * jnp.clip(yy2 - yy1, 0.0, None)
        iou = inter / (area_i + scr[4] - inter + 1e-8)
        sc_new = jnp.where(iou > 0.5, _NEG, sc)
        sc_new = jnp.where(lin == il, _NEG, sc_new)
        scr[11] = sc_new

        valid = m > jnp.float32(-1e30)
        kept = jnp.where(valid, sig_i, jnp.float32(-1.0))
        rmask = kept > 0.0
        row = jnp.zeros((1, 128), jnp.float32)
        row = jnp.where(lane == 0, ox1, row)
        row = jnp.where(lane == 1, oy1, row)
        row = jnp.where(lane == 2, ox2, row)
        row = jnp.where(lane == 3, oy2, row)
        row = jnp.where(lane == 4, kept, row)
        row = jnp.where(lane == 5, cls_i + 1.0, row)
        row = jnp.where(rmask, row, 0.0)
        o_ref[0, pl.ds(j, 1), :] = row
        return carry

    lax.fori_loop(0, 100, body, jnp.int32(0))


def kernel(cls_p3, cls_p4, cls_p5, cls_p6, cls_p7, box_p3, box_p4, box_p5, box_p6, box_p7, anchor_boxes, img_scales, img_size):
    cls_list = [cls_p3, cls_p4, cls_p5, cls_p6, cls_p7]
    box_list = [box_p3, box_p4, box_p5, box_p6, box_p7]
    B = cls_p3.shape[0]
    cls_all = jnp.concatenate(
        [jnp.transpose(cv, (0, 2, 3, 1)).reshape(B, -1, _NUM_CLASSES) for cv in cls_list], 1)
    box_all = jnp.concatenate(
        [jnp.transpose(bv, (0, 2, 3, 1)).reshape(B, -1, 4) for bv in box_list], 1)
    A = cls_all.shape[1]
    N = A * _NUM_CLASSES
    flat = cls_all.reshape(B, N)

    lcm = np.lcm(_CH1, _CH2)
    n_pad = int(-(-N // lcm) * lcm)
    flat_p = jnp.pad(flat, ((0, 0), (0, n_pad - N)), constant_values=float('-inf'))

    a_pad = int(-(-A // 128) * 128)
    boxT = jnp.pad(jnp.transpose(box_all, (0, 2, 1)), ((0, 0), (0, 0), (0, a_pad - A)))
    ancT = jnp.pad(anchor_boxes.T, ((0, 0), (0, a_pad - A)))
    t8 = jnp.concatenate([boxT, jnp.broadcast_to(ancT[None], (B, 4, a_pad))], axis=1)

    c1 = n_pad // _CH1
    f1 = flat_p.reshape(B, c1, 1, _CH1)
    ts, rn = pl.pallas_call(
        _k1_body,
        grid=(_NPASS, B, c1),
        in_specs=[pl.BlockSpec((1, 1, 1, _CH1), lambda p, b, c: (b, c, 0, 0))],
        out_specs=[pl.BlockSpec((1, 8, 128), lambda p, b, c: (b, 0, 0)),
                   pl.BlockSpec((1, 8, 128), lambda p, b, c: (b, 0, 0))],
        out_shape=[jax.ShapeDtypeStruct((B, 8, 128), jnp.int32),
                   jax.ShapeDtypeStruct((B, 8, 128), jnp.int32)],
        scratch_shapes=[pltpu.SMEM((_NBIN,), jnp.int32),
                        pltpu.SMEM((B,), jnp.int32),
                        pltpu.SMEM((B,), jnp.int32)],
    )(f1)

    c2 = n_pad // _CH2
    f2 = flat_p.reshape(B, c2, 1, _CH2)
    oidx, oval = pl.pallas_call(
        _k2_body,
        grid=(B, c2),
        in_specs=[pl.BlockSpec((1, 1, 1, _CH2), lambda b, c: (b, c, 0, 0)),
                  pl.BlockSpec((1, 8, 128), lambda b, c: (b, 0, 0)),
                  pl.BlockSpec((1, 8, 128), lambda b, c: (b, 0, 0))],
        out_specs=[pl.BlockSpec((1, 1, _W), lambda b, c: (b, 0, 0), memory_space=pltpu.SMEM),
                   pl.BlockSpec((1, 1, _W), lambda b, c: (b, 0, 0), memory_space=pltpu.SMEM)],
        out_shape=[jax.ShapeDtypeStruct((B, 1, _W), jnp.int32),
                   jax.ShapeDtypeStruct((B, 1, _W), jnp.float32)],
        scratch_shapes=[pltpu.SMEM((2,), jnp.int32)],
    )(f2, ts, rn)

    det = pl.pallas_call(
        functools.partial(_k3_body, a_pad),
        grid=(B,),
        in_specs=[pl.BlockSpec((1, 8, a_pad), lambda b: (b, 0, 0)),
                  pl.BlockSpec((1, 1, _W), lambda b: (b, 0, 0)),
                  pl.BlockSpec((1, 1, _W), lambda b: (b, 0, 0))],
        out_specs=pl.BlockSpec((1, 128, 128), lambda b: (b, 0, 0)),
        out_shape=jax.ShapeDtypeStruct((B, 128, 128), jnp.float32),
        scratch_shapes=[pltpu.VMEM((12, _W // 128, 128), jnp.float32)],
    )(t8, oval, oidx)

    out = det[:, :100, :6]
    lim = jnp.concatenate([img_size, img_size], axis=1) * img_scales[:, None]
    coords = jnp.clip(out[..., :4] * img_scales[:, None, None], 0.0, lim[:, None, :])
    return jnp.concatenate([coords, out[..., 4:6]], axis=-1)
